# baseline (speedup 1.0000x reference)
import functools

import jax
import jax.numpy as jnp
from jax import lax
from jax.experimental import pallas as pl
from jax.experimental.pallas import tpu as pltpu
from jax.experimental.pallas import tpu_sc as plsc

N = 10000
D = 128
R = 3
E = 320000

NC = 2
NS = 16
NW = NC * NS

NP = 10240
SP = NP // NS
BCH = 128
NB = SP // BCH

EA = E // NW
CA = 80
NCH_A = EA // CA

EC = E // NS
CC = 80
NCH_C = EC // CC

_mesh2 = plsc.VectorSubcoreMesh(core_axis_name="c", subcore_axis_name="s")


NPR = NP // BCH


@functools.partial(
    pl.kernel,
    out_type=jax.ShapeDtypeStruct((NW, R, NPR, BCH), jnp.float32),
    mesh=_mesh2,
    scratch_types=[
        pltpu.VMEM((NCH_A, CA), jnp.int32),
        pltpu.VMEM((R * NPR, BCH), jnp.float32),
    ],
    compiler_params=pltpu.CompilerParams(needs_layout_passes=False),
)
def _deg_kernel(dstc_hbm, out_hbm, idxv, hist):
    c = lax.axis_index("c")
    s = lax.axis_index("s")
    w = c * NS + s

    zero16 = jnp.zeros((16,), jnp.float32)
    ones16 = jnp.ones((16,), jnp.float32)

    def zero_row(i, carry):
        for k in range(BCH // 16):
            hist[i, pl.ds(k * 16, 16)] = zero16
        return carry

    lax.fori_loop(0, R * NPR, zero_row, 0)

    for r in range(R):
        pltpu.sync_copy(dstc_hbm.at[r, w], idxv)

        def edge_body(j, carry):
            for k in range(CA // 16):
                d = idxv[j, pl.ds(k * 16, 16)]
                plsc.addupdate_scatter(
                    hist, [(d >> 7) + r * NPR, d & 127], ones16)
            return carry

        lax.fori_loop(0, NCH_A, edge_body, 0)

    for r in range(R):
        pltpu.sync_copy(hist.at[pl.ds(r * NPR, NPR)], out_hbm.at[w, r])


def _dis_body(parts_ref, dis_ref):
    parts = parts_ref[...]
    deg = jnp.sum(parts, axis=0) + 1.0
    dis_ref[...] = lax.rsqrt(deg)


def _dis_stage(parts):
    return pl.pallas_call(
        _dis_body,
        grid=(1,),
        in_specs=[pl.BlockSpec((NW, R, NPR, BCH), lambda i: (0, 0, 0, 0))],
        out_specs=pl.BlockSpec((R, NPR, BCH), lambda i: (0, 0, 0)),
        out_shape=jax.ShapeDtypeStruct((R, NPR, BCH), jnp.float32),
    )(parts)


BN = 400
NBLK = N // BN


def _mm_body(dis_ref, x_ref, w_ref, h_ref, hp_ref):
    dis = dis_ref[...][:, 0, 0, :]
    xb = x_ref[...]
    for r in range(R):
        h = jnp.dot(xb, w_ref[r], preferred_element_type=jnp.float32)
        h_ref[r, :, :] = h
        hp_ref[r, :, :] = h * dis[r][:, None]


def _mm_stage(dis4, x, W):
    big = pl.BlockSpec((R, BN, D), lambda i: (0, i, 0))
    big_shape = jax.ShapeDtypeStruct((R, N, D), jnp.float32)
    return pl.pallas_call(
        _mm_body,
        grid=(NBLK,),
        in_specs=[
            pl.BlockSpec((R, 1, 1, BN), lambda i: (0, i, 0, 0)),
            pl.BlockSpec((BN, D), lambda i: (i, 0)),
            pl.BlockSpec((R, D, D), lambda i: (0, 0, 0)),
        ],
        out_specs=[big, big],
        out_shape=[big_shape, big_shape],
    )(dis4, x, W)


@functools.partial(
    pl.kernel,
    out_type=jax.ShapeDtypeStruct((R * NP, D), jnp.float32),
    mesh=_mesh2,
    scratch_types=[
        pltpu.VMEM((NB, BCH), jnp.int32),
        pltpu.VMEM((BCH, D), jnp.float32),
        pltpu.SemaphoreType.DMA,
    ],
)
def _perm_kernel(h_hbm, permo_hbm, hq_hbm, pidxv, buf, sem):
    c = lax.axis_index("c")
    s = lax.axis_index("s")
    w = c * NS + s

    def do_unit(u):
        r_t = u // NS
        s_t = u % NS
        pltpu.sync_copy(permo_hbm.at[r_t, s_t], pidxv)
        for j in range(NB):
            pltpu.async_copy(h_hbm.at[pidxv.at[j]], buf, sem).wait()
            off = pl.multiple_of(r_t * NP + s_t * SP + j * BCH, BCH)
            pltpu.sync_copy(buf, hq_hbm.at[pl.ds(off, BCH)])

    do_unit(w)

    @pl.when(w < R * NS - NW)
    def _extra():
        do_unit(w + NW)


def _scale_body(hperm_ref, dis_ref, hq_ref):
    hq_ref[...] = hperm_ref[...] * dis_ref[...][:, 0, 0, :, None]


def _scale_stage(hperm, dis4):
    blk = pl.BlockSpec((1, BN, D), lambda r, i: (r, i, 0))
    return pl.pallas_call(
        _scale_body,
        grid=(R, NBLK),
        in_specs=[
            blk,
            pl.BlockSpec((1, 1, 1, BN), lambda r, i: (r, i, 0, 0)),
        ],
        out_specs=blk,
        out_shape=jax.ShapeDtypeStruct((R, NP, D), jnp.float32),
    )(hperm, dis4)


HN = NP // 2
TRW = 5120
SPH = HN // NS


@functools.partial(
    pl.kernel,
    out_type=(
        jax.ShapeDtypeStruct((R, 2, HN, D), jnp.float32),
        jax.ShapeDtypeStruct((R, 2, HN, D), jnp.float32),
    ),
    mesh=_mesh2,
    scratch_types=[
        pltpu.VMEM((NCH_C, CC), jnp.int32),
        pltpu.VMEM((NCH_C, CC), jnp.int32),
        pltpu.VMEM((CC, D), jnp.float32),
        pltpu.VMEM((64, D), jnp.float32),
        pltpu.SemaphoreType.DMA,
        pltpu.VMEM_SHARED((TRW + BCH, D), jnp.float32),
    ],
)
def _sc_kernel(hp_hbm, hq_hbm, srcr_hbm, dstl_hbm, accp_hbm, accq_hbm,
               srcv, dstv, buf, zdbuf, sem, acc):
    c = lax.axis_index("c")
    s = lax.axis_index("s")
    hbase = pl.multiple_of(s * SPH, SPH)

    zero16 = jnp.zeros((16,), jnp.float32)

    def zero_zdbuf():
        def zrow(i, carry):
            for k in range(D // 16):
                zdbuf[i, pl.ds(k * 16, 16)] = zero16
            return carry

        lax.fori_loop(0, 64, zrow, 0)

    def remap_idx(tbl_off, half):
        lo = half * HN

        def rbody(j, carry):
            for k in range(CC // 16):
                sl = pl.ds(k * 16, 16)
                dv = dstv[j, sl]
                inh = jnp.logical_and(dv >= lo, dv < lo + HN)
                dstv[j, sl] = jnp.where(inh, dv - lo, TRW)
                sv = srcv[j, sl]
                srcv[j, sl] = jnp.where(inh, sv, 0) + jnp.int32(tbl_off)
            return carry

        lax.fori_loop(0, NCH_C, rbody, 0)

    def edge_phase(tbl_hbm, r, half, tbl_off):
        pltpu.sync_copy(srcr_hbm.at[r, s], srcv)
        pltpu.sync_copy(dstl_hbm.at[r, s], dstv)
        remap_idx(tbl_off, half)

        def body(j, carry):
            pltpu.async_copy(tbl_hbm.at[srcv.at[j]], buf, sem).wait()
            pltpu.sync_copy(buf, acc.at[dstv.at[j]], add=True)
            return carry

        lax.fori_loop(0, NCH_C, body, 0)

    def dump_phase(out_hbm, r, half):
        for j in range(SPH // 64):
            sl = pl.ds(hbase + j * 64, 64)
            pltpu.sync_copy(acc.at[sl], zdbuf)
            pltpu.sync_copy(zdbuf, out_hbm.at[r, half, sl])

    for r in range(R):
        for half in range(2):
            zero_zdbuf()
            for j in range(SPH // 64):
                pltpu.sync_copy(zdbuf, acc.at[pl.ds(hbase + j * 64, 64)])
            plsc.subcore_barrier()

            @pl.when(c == 0)
            def _pos():
                edge_phase(hp_hbm, r, half, r * N)

            @pl.when(c == 1)
            def _neg():
                edge_phase(hq_hbm, r, half, r * NP)

            plsc.subcore_barrier()

            @pl.when(c == 0)
            def _dump_pos():
                dump_phase(accp_hbm, r, half)

            @pl.when(c == 1)
            def _dump_neg():
                dump_phase(accq_hbm, r, half)

            plsc.subcore_barrier()


def _fin_body(accp_ref, accq_ref, hp_ref, hq_ref, dis_ref, b_ref,
              pos_ref, neg_ref, sum_ref):
    i = pl.program_id(1)
    dis = dis_ref[...][:, 0, 0, :, None]
    bb = b_ref[...]
    pos = jnp.maximum(dis * (accp_ref[...] + hp_ref[...]) + bb, 0.0)
    neg = jnp.maximum(dis * (accq_ref[...] + hq_ref[...]) + bb, 0.0)
    pos_ref[...] = pos
    neg_ref[...] = neg

    @pl.when(i == 0)
    def _init():
        sum_ref[...] = jnp.zeros_like(sum_ref)

    sum_ref[...] += jnp.sum(pos, axis=1, keepdims=True)

    @pl.when(i == NBLK - 1)
    def _final():
        sum_ref[...] = sum_ref[...] * (1.0 / N)


def _fin_stage(accp, accq, hp, hq, dis, b):
    blk = pl.BlockSpec((1, BN, D), lambda r, i: (r, i, 0))
    return pl.pallas_call(
        _fin_body,
        grid=(R, NBLK),
        in_specs=[
            blk, blk, blk, blk,
            pl.BlockSpec((1, 1, 1, BN), lambda r, i: (r, i, 0, 0)),
            pl.BlockSpec((1, 1, D), lambda r, i: (r, 0, 0)),
        ],
        out_specs=[
            blk, blk,
            pl.BlockSpec((1, 1, D), lambda r, i: (r, 0, 0)),
        ],
        out_shape=[
            jax.ShapeDtypeStruct((R, N, D), jnp.float32),
            jax.ShapeDtypeStruct((R, N, D), jnp.float32),
            jax.ShapeDtypeStruct((R, 1, D), jnp.float32),
        ],
    )(accp, accq, hp, hq, dis, b.reshape(R, 1, D))


_DEBUG_JNP_C2 = False


def kernel(x, edge_index, W, b, perm, dropout_probability):
    del dropout_probability
    src = edge_index[:, 0, :].astype(jnp.int32)
    dst = edge_index[:, 1, :].astype(jnp.int32)
    roffN = (jnp.arange(R, dtype=jnp.int32) * N)[:, None]

    dstc = dst.reshape(R, NW, NCH_A, CA)
    degparts = _deg_kernel(dstc)
    dis3 = _dis_stage(degparts)
    dis4 = dis3.reshape(R, NP)[:, :N].reshape(R, NBLK, 1, BN)

    h, hp = _mm_stage(dis4, x, W)

    permo = (jnp.concatenate(
        [perm.astype(jnp.int32), jnp.zeros((R, NP - N), jnp.int32)],
        axis=1) + roffN).reshape(R, NS, NB, BCH)

    hperm = _perm_kernel(h.reshape(R * N, D), permo)
    hq = _scale_stage(hperm.reshape(R, NP, D), dis4)

    srcr = src.reshape(R, NS, NCH_C, CC)
    dstl = dst.reshape(R, NS, NCH_C, CC)

    if _DEBUG_JNP_C2:
        hqf = hq.reshape(R * NP, D)
        hpf = hp.reshape(R * N, D)
        accp3 = jnp.stack([
            jnp.zeros((N, D), jnp.float32).at[dst[r]].add(
                hpf[src[r] + r * N]) for r in range(R)])
        accq3 = jnp.stack([
            jnp.zeros((N, D), jnp.float32).at[dst[r]].add(
                hqf[src[r] + r * NP]) for r in range(R)])
        pad = jnp.zeros((R, NP - N, D), jnp.float32)
        accp = jnp.concatenate([accp3, pad], axis=1).reshape(R, 2, HN, D)
        accq = jnp.concatenate([accq3, pad], axis=1).reshape(R, 2, HN, D)
    else:
        accp, accq = _sc_kernel(
            hp.reshape(R * N, D), hq.reshape(R * NP, D), srcr, dstl)

    pos, neg, ssum = _fin_stage(
        accp.reshape(R, NP, D), accq.reshape(R, NP, D), hp, hq, dis4, b)
    return pos, neg, ssum
